# Initial kernel scaffold; baseline (speedup 1.0000x reference)
import jax
import jax.numpy as jnp
from jax.experimental import pallas as pl


def kernel(x, edge_index, batch, W1, b1, W2, b2, W3, b3, Wf, bf):
    raise NotImplementedError("write your pallas kernel here")


# speedup vs baseline: 14.9449x; 14.9449x over previous
import functools

import jax
import jax.numpy as jnp
from jax import lax
from jax.experimental import pallas as pl
from jax.experimental.pallas import tpu as pltpu
from jax.experimental.pallas import tpu_sc as plsc

_N = 10000
_E = 320000
_G = 64

_NC = 2
_NS = 16
_NPAD = 10112
_RPT = _NPAD // _NS
_CHUNK = 128
_NCHUNKS = _E // _CHUNK
_CH_PER_CORE = _NCHUNKS // _NC
_T_STEPS = -(-_CH_PER_CORE // _NS)

_mesh = plsc.VectorSubcoreMesh(core_axis_name="c", subcore_axis_name="s")


_DEGW = 16


@functools.partial(
    pl.kernel,
    out_type=jax.ShapeDtypeStruct((_NC, _NPAD, _DEGW), jnp.float32),
    mesh=_mesh,
    scratch_types=[
        pltpu.VMEM_SHARED((_NPAD, _DEGW), jnp.float32),
        pltpu.VMEM((_CHUNK, _DEGW), jnp.float32),
        pltpu.VMEM((_CHUNK,), jnp.int32),
    ],
    name="gcn_deg",
    compiler_params=pltpu.CompilerParams(use_tc_tiling_on_sc=False),
)
def _deg_sc(dst_hbm, ones_hbm, zeros_hbm, out_hbm, acc, ones_v, didx):
    c = lax.axis_index("c")
    s = lax.axis_index("s")
    r0 = s * _RPT
    pltpu.sync_copy(zeros_hbm.at[pl.ds(r0, _RPT)], acc.at[pl.ds(r0, _RPT)])
    pltpu.sync_copy(ones_hbm, ones_v)
    plsc.subcore_barrier()

    def body(t, carry):
        j = s + _NS * t

        @pl.when(j < _CH_PER_CORE)
        def _():
            e0 = (c * _CH_PER_CORE + j) * _CHUNK
            pltpu.sync_copy(dst_hbm.at[pl.ds(e0, _CHUNK)], didx)
            pltpu.sync_copy(ones_v, acc.at[didx], add=True)

        return carry

    lax.fori_loop(0, _T_STEPS, body, 0)
    plsc.subcore_barrier()
    pltpu.sync_copy(acc.at[pl.ds(r0, _RPT)], out_hbm.at[c, pl.ds(r0, _RPT)])


def _make_agg(F):
    @functools.partial(
        pl.kernel,
        out_type=jax.ShapeDtypeStruct((_NC, _NPAD, F), jnp.float32),
        mesh=_mesh,
        scratch_types=[
            pltpu.VMEM_SHARED((_NPAD, F), jnp.float32),
            pltpu.VMEM((_CHUNK, F), jnp.float32),
            pltpu.VMEM((_CHUNK,), jnp.int32),
            pltpu.VMEM((_CHUNK,), jnp.int32),
            pltpu.SemaphoreType.DMA,
        ],
        name=f"gcn_agg_f{F}",
        compiler_params=pltpu.CompilerParams(use_tc_tiling_on_sc=False),
    )
    def agg(g_hbm, src_hbm, dst_hbm, zeros_hbm, out_hbm, acc, rows, sidx, didx, sem):
        c = lax.axis_index("c")
        s = lax.axis_index("s")
        r0 = s * _RPT
        pltpu.sync_copy(zeros_hbm.at[pl.ds(r0, _RPT)], acc.at[pl.ds(r0, _RPT)])
        plsc.subcore_barrier()

        def body(t, carry):
            j = s + _NS * t

            @pl.when(j < _CH_PER_CORE)
            def _():
                e0 = (c * _CH_PER_CORE + j) * _CHUNK
                pltpu.sync_copy(src_hbm.at[pl.ds(e0, _CHUNK)], sidx)
                pltpu.sync_copy(dst_hbm.at[pl.ds(e0, _CHUNK)], didx)
                pltpu.async_copy(g_hbm.at[sidx], rows, sem).wait()
                pltpu.sync_copy(rows, acc.at[didx], add=True)

            return carry

        lax.fori_loop(0, _T_STEPS, body, 0)
        plsc.subcore_barrier()
        pltpu.sync_copy(acc.at[pl.ds(r0, _RPT)], out_hbm.at[c, pl.ds(r0, _RPT)])

    return agg


_agg128 = _make_agg(128)
_agg64 = _make_agg(64)


_BR = 2000
_DBR = 1000


def _b1_body(x_ref, w_ref, d0_ref, d1_ref, g_ref, dinv_ref):
    deg = d0_ref[...] + d1_ref[...] + 1.0
    dinv = lax.rsqrt(deg)
    h = jnp.dot(x_ref[...], w_ref[...], preferred_element_type=jnp.float32)
    g_ref[...] = h * dinv
    dinv_ref[...] = dinv


def _b1(x, w1, d0, d1):
    nb = _N // _BR
    return pl.pallas_call(
        _b1_body,
        grid=(nb,),
        in_specs=[
            pl.BlockSpec((_BR, 128), lambda i: (i, 0)),
            pl.BlockSpec((128, 128), lambda i: (0, 0)),
            pl.BlockSpec((_BR, 1), lambda i: (i, 0)),
            pl.BlockSpec((_BR, 1), lambda i: (i, 0)),
        ],
        out_specs=[
            pl.BlockSpec((_BR, 128), lambda i: (i, 0)),
            pl.BlockSpec((_BR, 1), lambda i: (i, 0)),
        ],
        out_shape=[
            jax.ShapeDtypeStruct((_N, 128), jnp.float32),
            jax.ShapeDtypeStruct((_N, 1), jnp.float32),
        ],
    )(x, w1, d0, d1)


def _bmid_body(a0_ref, a1_ref, g_ref, dinv_ref, b_ref, w_ref, out_ref):
    pre = (a0_ref[...] + a1_ref[...] + g_ref[...]) * dinv_ref[...] + b_ref[...]
    h = jnp.maximum(pre, 0.0)
    out_ref[...] = jnp.dot(h, w_ref[...], preferred_element_type=jnp.float32) * dinv_ref[...]


def _bmid(a0, a1, g, dinv, b, w, f_in, f_out):
    nb = _N // _BR
    return pl.pallas_call(
        _bmid_body,
        grid=(nb,),
        in_specs=[
            pl.BlockSpec((_BR, f_in), lambda i: (i, 0)),
            pl.BlockSpec((_BR, f_in), lambda i: (i, 0)),
            pl.BlockSpec((_BR, f_in), lambda i: (i, 0)),
            pl.BlockSpec((_BR, 1), lambda i: (i, 0)),
            pl.BlockSpec((1, f_in), lambda i: (0, 0)),
            pl.BlockSpec((f_in, f_out), lambda i: (0, 0)),
        ],
        out_specs=pl.BlockSpec((_BR, f_out), lambda i: (i, 0)),
        out_shape=jax.ShapeDtypeStruct((_N, f_out), jnp.float32),
    )(a0, a1, g, dinv, b, w)


def _pool_body(a0_ref, a1_ref, g_ref, dinv_ref, b_ref, batch_ref, wf_ref, bf_ref,
               out_ref, sums, cnt):
    i = pl.program_id(0)
    nb = pl.num_programs(0)
    h = (a0_ref[...] + a1_ref[...] + g_ref[...]) * dinv_ref[...] + b_ref[...]
    m = (batch_ref[...] == lax.broadcasted_iota(jnp.int32, (_DBR, _G), 1)
         ).astype(jnp.float32)
    ssum = lax.dot_general(m, h, (((0,), (0,)), ((), ())),
                           preferred_element_type=jnp.float32)
    csum = lax.dot_general(m, jnp.ones((_DBR, 1), jnp.float32),
                           (((0,), (0,)), ((), ())),
                           preferred_element_type=jnp.float32)

    @pl.when(i == 0)
    def _():
        sums[...] = jnp.zeros_like(sums)
        cnt[...] = jnp.zeros_like(cnt)

    sums[...] += ssum
    cnt[...] += csum

    @pl.when(i == nb - 1)
    def _():
        pooled = sums[...] / jnp.maximum(cnt[...], 1.0)
        sg = jax.nn.sigmoid(pooled)
        out_ref[...] = jnp.dot(sg, wf_ref[...],
                               preferred_element_type=jnp.float32) + bf_ref[...]


def _pool(a0, a1, g, dinv, b, batch2d, wf, bf2d):
    nb = _N // _DBR
    return pl.pallas_call(
        _pool_body,
        grid=(nb,),
        in_specs=[
            pl.BlockSpec((_DBR, 64), lambda i: (i, 0)),
            pl.BlockSpec((_DBR, 64), lambda i: (i, 0)),
            pl.BlockSpec((_DBR, 64), lambda i: (i, 0)),
            pl.BlockSpec((_DBR, 1), lambda i: (i, 0)),
            pl.BlockSpec((1, 64), lambda i: (0, 0)),
            pl.BlockSpec((_DBR, 1), lambda i: (i, 0)),
            pl.BlockSpec((64, 1), lambda i: (0, 0)),
            pl.BlockSpec((1, 1), lambda i: (0, 0)),
        ],
        out_specs=pl.BlockSpec((_G, 1), lambda i: (0, 0)),
        out_shape=jax.ShapeDtypeStruct((_G, 1), jnp.float32),
        scratch_shapes=[
            pltpu.VMEM((_G, 64), jnp.float32),
            pltpu.VMEM((_G, 1), jnp.float32),
        ],
    )(a0, a1, g, dinv, b, batch2d, wf, bf2d)


def kernel(x, edge_index, batch, W1, b1, W2, b2, W3, b3, Wf, bf):
    src = edge_index[0]
    dst = edge_index[1]

    zeros128 = jnp.zeros((_NPAD, 128), jnp.float32)
    zeros64 = jnp.zeros((_NPAD, 64), jnp.float32)
    zerosw = jnp.zeros((_NPAD, _DEGW), jnp.float32)
    ones_chunk = jnp.ones((_CHUNK, _DEGW), jnp.float32)

    degp = _deg_sc(dst, ones_chunk, zerosw)
    d0 = degp[0, :_N, 0:1]
    d1 = degp[1, :_N, 0:1]

    g1, dinv = _b1(x, W1, d0, d1)
    a1 = _agg128(g1, src, dst, zeros128)
    g2 = _bmid(a1[0, :_N], a1[1, :_N], g1, dinv,
               b1.reshape(1, 128), W2, 128, 64)
    a2 = _agg64(g2, src, dst, zeros64)
    g3 = _bmid(a2[0, :_N], a2[1, :_N], g2, dinv,
               b2.reshape(1, 64), W3, 64, 64)
    a3 = _agg64(g3, src, dst, zeros64)
    out = _pool(a3[0, :_N], a3[1, :_N], g3, dinv,
                b3.reshape(1, 64), batch.reshape(_N, 1), Wf, bf.reshape(1, 1))
    return out


# speedup vs baseline: 28.4139x; 1.9012x over previous
import functools

import jax
import jax.numpy as jnp
from jax import lax
from jax.experimental import pallas as pl
from jax.experimental.pallas import tpu as pltpu
from jax.experimental.pallas import tpu_sc as plsc

_N = 10000
_E = 320000
_G = 64

_NC = 2
_NS = 16
_NPAD = 10112
_RPT = _NPAD // _NS
_CHUNK = 128
_NCHUNKS = _E // _CHUNK
_CH_PER_CORE = _NCHUNKS // _NC
_T_STEPS = -(-_CH_PER_CORE // _NS)

_mesh = plsc.VectorSubcoreMesh(core_axis_name="c", subcore_axis_name="s")

_TMAX = 79
_NCHPAD = 2528
_NB = 4


_DEGW = 16


@functools.partial(
    pl.kernel,
    out_type=jax.ShapeDtypeStruct((_NC, _NPAD, _DEGW), jnp.float32),
    mesh=_mesh,
    scratch_types=[
        pltpu.VMEM_SHARED((_NPAD, _DEGW), jnp.float32),
        pltpu.VMEM((_CHUNK, _DEGW), jnp.float32),
        pltpu.VMEM((_TMAX, _CHUNK), jnp.int32),
        pltpu.SemaphoreType.DMA,
    ],
    name="gcn_deg",
    compiler_params=pltpu.CompilerParams(use_tc_tiling_on_sc=False),
)
def _deg_sc(dst2d_hbm, ones_hbm, zeros_hbm, out_hbm, acc, ones_v, didx2d, sem):
    c = lax.axis_index("c")
    s = lax.axis_index("s")
    w = c * _NS + s
    r0 = s * _RPT
    pltpu.sync_copy(zeros_hbm.at[pl.ds(r0, _RPT)], acc.at[pl.ds(r0, _RPT)])
    pltpu.sync_copy(ones_hbm, ones_v)
    base = 78 * w + jnp.minimum(w, 4)
    cnt = jnp.where(w < 4, 79, 78)
    pltpu.sync_copy(dst2d_hbm.at[pl.ds(base, _TMAX)], didx2d)
    plsc.subcore_barrier()

    def body(grp, carry):
        for b in range(_NB):
            t = grp * _NB + b

            @pl.when(t < cnt)
            def _():
                pltpu.async_copy(ones_v, acc.at[didx2d.at[t]], sem, add=True)

        for b in range(_NB):
            t = grp * _NB + b

            @pl.when(t < cnt)
            def _():
                pltpu.make_async_copy(ones_v, acc.at[didx2d.at[0]], sem).wait()

        return carry

    lax.fori_loop(0, -(-_TMAX // _NB), body, 0)
    plsc.subcore_barrier()
    pltpu.sync_copy(acc.at[pl.ds(r0, _RPT)], out_hbm.at[c, pl.ds(r0, _RPT)])


def _make_agg(F):
    @functools.partial(
        pl.kernel,
        out_type=jax.ShapeDtypeStruct((_NC, _NPAD, F), jnp.float32),
        mesh=_mesh,
        scratch_types=(
            [pltpu.VMEM_SHARED((_NPAD, F), jnp.float32)]
            + [pltpu.VMEM((_CHUNK, F), jnp.float32) for _ in range(_NB)]
            + [pltpu.VMEM((_TMAX, _CHUNK), jnp.int32),
               pltpu.VMEM((_TMAX, _CHUNK), jnp.int32)]
            + [pltpu.SemaphoreType.DMA for _ in range(2 * _NB)]
        ),
        name=f"gcn_agg_f{F}",
        compiler_params=pltpu.CompilerParams(use_tc_tiling_on_sc=False),
    )
    def agg(g_hbm, src2d_hbm, dst2d_hbm, zeros_hbm, out_hbm, acc,
            r0buf, r1buf, r2buf, r3buf, sidx2d, didx2d,
            g0, g1, g2, g3, s0, s1, s2, s3):
        rows = [r0buf, r1buf, r2buf, r3buf]
        gsem = [g0, g1, g2, g3]
        ssem = [s0, s1, s2, s3]
        c = lax.axis_index("c")
        s = lax.axis_index("s")
        w = c * _NS + s
        r0 = s * _RPT
        pltpu.sync_copy(zeros_hbm.at[pl.ds(r0, _RPT)], acc.at[pl.ds(r0, _RPT)])

        base = 78 * w + jnp.minimum(w, 4)
        cnt = jnp.where(w < 4, 79, 78)
        pltpu.sync_copy(src2d_hbm.at[pl.ds(base, _TMAX)], sidx2d)
        pltpu.sync_copy(dst2d_hbm.at[pl.ds(base, _TMAX)], didx2d)
        plsc.subcore_barrier()

        def issue_gather(b, t):
            pltpu.async_copy(g_hbm.at[sidx2d.at[t]], rows[b], gsem[b])

        def wait_gather(b):
            pltpu.make_async_copy(g_hbm.at[sidx2d.at[0]], rows[b], gsem[b]).wait()

        def issue_scat(b, t):
            pltpu.async_copy(rows[b], acc.at[didx2d.at[t]], ssem[b], add=True)

        def wait_scat(b):
            pltpu.make_async_copy(rows[b], acc.at[didx2d.at[0]], ssem[b]).wait()

        for b in range(_NB):
            issue_gather(b, b)

        def body(grp, carry):
            for b in range(_NB):
                t = grp * _NB + b

                @pl.when(t < cnt)
                def _():
                    wait_gather(b)
                    issue_scat(b, t)

            for b in range(_NB):
                t = grp * _NB + b

                @pl.when(t + _NB < cnt)
                def _():
                    wait_scat(b)
                    issue_gather(b, t + _NB)

            return carry

        lax.fori_loop(0, -(-_TMAX // _NB), body, 0)
        for b in range(_NB):
            wait_scat(b)
        plsc.subcore_barrier()
        pltpu.sync_copy(acc.at[pl.ds(r0, _RPT)], out_hbm.at[c, pl.ds(r0, _RPT)])

    return agg


_agg64 = _make_agg(64)


_BR = 2000
_DBR = 1000


def _b1_body(x_ref, w_ref, d0_ref, d1_ref, gl_ref, gr_ref, dinv_ref):
    deg = d0_ref[...] + d1_ref[...] + 1.0
    dinv = lax.rsqrt(deg)
    h = jnp.dot(x_ref[...], w_ref[...], preferred_element_type=jnp.float32)
    g = h * dinv
    gl_ref[...] = g[:, :64]
    gr_ref[...] = g[:, 64:]
    dinv_ref[...] = dinv


def _b1(x, w1, d0, d1):
    nb = _N // _BR
    return pl.pallas_call(
        _b1_body,
        grid=(nb,),
        in_specs=[
            pl.BlockSpec((_BR, 128), lambda i: (i, 0)),
            pl.BlockSpec((128, 128), lambda i: (0, 0)),
            pl.BlockSpec((_BR, 1), lambda i: (i, 0)),
            pl.BlockSpec((_BR, 1), lambda i: (i, 0)),
        ],
        out_specs=[
            pl.BlockSpec((_BR, 64), lambda i: (i, 0)),
            pl.BlockSpec((_BR, 64), lambda i: (i, 0)),
            pl.BlockSpec((_BR, 1), lambda i: (i, 0)),
        ],
        out_shape=[
            jax.ShapeDtypeStruct((_N, 64), jnp.float32),
            jax.ShapeDtypeStruct((_N, 64), jnp.float32),
            jax.ShapeDtypeStruct((_N, 1), jnp.float32),
        ],
    )(x, w1, d0, d1)


def _b2_body(a0l_ref, a1l_ref, a0r_ref, a1r_ref, gl_ref, gr_ref, dinv_ref,
             bl_ref, br_ref, wl_ref, wr_ref, out_ref):
    dinv = dinv_ref[...]
    hl = jnp.maximum((a0l_ref[...] + a1l_ref[...] + gl_ref[...]) * dinv + bl_ref[...], 0.0)
    hr = jnp.maximum((a0r_ref[...] + a1r_ref[...] + gr_ref[...]) * dinv + br_ref[...], 0.0)
    h = (jnp.dot(hl, wl_ref[...], preferred_element_type=jnp.float32)
         + jnp.dot(hr, wr_ref[...], preferred_element_type=jnp.float32))
    out_ref[...] = h * dinv


def _b2(a0l, a1l, a0r, a1r, gl, gr, dinv, b1v, w2):
    nb = _N // _BR
    row = pl.BlockSpec((_BR, 64), lambda i: (i, 0))
    return pl.pallas_call(
        _b2_body,
        grid=(nb,),
        in_specs=[
            row, row, row, row, row, row,
            pl.BlockSpec((_BR, 1), lambda i: (i, 0)),
            pl.BlockSpec((1, 64), lambda i: (0, 0)),
            pl.BlockSpec((1, 64), lambda i: (0, 0)),
            pl.BlockSpec((64, 64), lambda i: (0, 0)),
            pl.BlockSpec((64, 64), lambda i: (0, 0)),
        ],
        out_specs=row,
        out_shape=jax.ShapeDtypeStruct((_N, 64), jnp.float32),
    )(a0l, a1l, a0r, a1r, gl, gr, dinv,
      b1v[:64].reshape(1, 64), b1v[64:].reshape(1, 64),
      w2[:64], w2[64:])


def _b3_body(a0_ref, a1_ref, g_ref, dinv_ref, b_ref, w_ref, out_ref):
    pre = (a0_ref[...] + a1_ref[...] + g_ref[...]) * dinv_ref[...] + b_ref[...]
    h = jnp.maximum(pre, 0.0)
    out_ref[...] = jnp.dot(h, w_ref[...], preferred_element_type=jnp.float32) * dinv_ref[...]


def _b3(a0, a1, g, dinv, b, w):
    nb = _N // _BR
    row = pl.BlockSpec((_BR, 64), lambda i: (i, 0))
    return pl.pallas_call(
        _b3_body,
        grid=(nb,),
        in_specs=[
            row, row, row,
            pl.BlockSpec((_BR, 1), lambda i: (i, 0)),
            pl.BlockSpec((1, 64), lambda i: (0, 0)),
            pl.BlockSpec((64, 64), lambda i: (0, 0)),
        ],
        out_specs=row,
        out_shape=jax.ShapeDtypeStruct((_N, 64), jnp.float32),
    )(a0, a1, g, dinv, b, w)


def _pool_body(a0_ref, a1_ref, g_ref, dinv_ref, b_ref, batch_ref, wf_ref, bf_ref,
               out_ref, sums, cnt):
    i = pl.program_id(0)
    nb = pl.num_programs(0)
    h = (a0_ref[...] + a1_ref[...] + g_ref[...]) * dinv_ref[...] + b_ref[...]
    m = (batch_ref[...] == lax.broadcasted_iota(jnp.int32, (_DBR, _G), 1)
         ).astype(jnp.float32)
    ssum = lax.dot_general(m, h, (((0,), (0,)), ((), ())),
                           preferred_element_type=jnp.float32)
    csum = lax.dot_general(m, jnp.ones((_DBR, 1), jnp.float32),
                           (((0,), (0,)), ((), ())),
                           preferred_element_type=jnp.float32)

    @pl.when(i == 0)
    def _():
        sums[...] = jnp.zeros_like(sums)
        cnt[...] = jnp.zeros_like(cnt)

    sums[...] += ssum
    cnt[...] += csum

    @pl.when(i == nb - 1)
    def _():
        pooled = sums[...] / jnp.maximum(cnt[...], 1.0)
        sg = jax.nn.sigmoid(pooled)
        out_ref[...] = jnp.dot(sg, wf_ref[...],
                               preferred_element_type=jnp.float32) + bf_ref[...]


def _pool(a0, a1, g, dinv, b, batch2d, wf, bf2d):
    nb = _N // _DBR
    return pl.pallas_call(
        _pool_body,
        grid=(nb,),
        in_specs=[
            pl.BlockSpec((_DBR, 64), lambda i: (i, 0)),
            pl.BlockSpec((_DBR, 64), lambda i: (i, 0)),
            pl.BlockSpec((_DBR, 64), lambda i: (i, 0)),
            pl.BlockSpec((_DBR, 1), lambda i: (i, 0)),
            pl.BlockSpec((1, 64), lambda i: (0, 0)),
            pl.BlockSpec((_DBR, 1), lambda i: (i, 0)),
            pl.BlockSpec((64, 1), lambda i: (0, 0)),
            pl.BlockSpec((1, 1), lambda i: (0, 0)),
        ],
        out_specs=pl.BlockSpec((_G, 1), lambda i: (0, 0)),
        out_shape=jax.ShapeDtypeStruct((_G, 1), jnp.float32),
        scratch_shapes=[
            pltpu.VMEM((_G, 64), jnp.float32),
            pltpu.VMEM((_G, 1), jnp.float32),
        ],
    )(a0, a1, g, dinv, b, batch2d, wf, bf2d)


def kernel(x, edge_index, batch, W1, b1, W2, b2, W3, b3, Wf, bf):
    pad = jnp.zeros((_NCHPAD * _CHUNK - _E,), jnp.int32)
    src2d = jnp.concatenate([edge_index[0], pad]).reshape(_NCHPAD, _CHUNK)
    dst2d = jnp.concatenate([edge_index[1], pad]).reshape(_NCHPAD, _CHUNK)

    zeros64 = jnp.zeros((_NPAD, 64), jnp.float32)
    zerosw = jnp.zeros((_NPAD, _DEGW), jnp.float32)
    ones_chunk = jnp.ones((_CHUNK, _DEGW), jnp.float32)

    degp = _deg_sc(dst2d, ones_chunk, zerosw)
    d0 = degp[0, :_N, 0:1]
    d1 = degp[1, :_N, 0:1]

    gl, gr, dinv = _b1(x, W1, d0, d1)
    a1l = _agg64(gl, src2d, dst2d, zeros64)
    a1r = _agg64(gr, src2d, dst2d, zeros64)
    g2 = _b2(a1l[0, :_N], a1l[1, :_N], a1r[0, :_N], a1r[1, :_N],
             gl, gr, dinv, b1, W2)
    a2 = _agg64(g2, src2d, dst2d, zeros64)
    g3 = _b3(a2[0, :_N], a2[1, :_N], g2, dinv,
             b2.reshape(1, 64), W3)
    a3 = _agg64(g3, src2d, dst2d, zeros64)
    out = _pool(a3[0, :_N], a3[1, :_N], g3, dinv,
                b3.reshape(1, 64), batch.reshape(_N, 1), Wf, bf.reshape(1, 1))
    return out
